# baseline (speedup 1.0000x reference)
import functools

import jax
import jax.numpy as jnp
from jax import lax
from jax.experimental import pallas as pl
from jax.experimental.pallas import tpu as pltpu
from jax.experimental.pallas import tpu_sc as plsc

_NC = 2
_NS = 16
_NW = _NC * _NS


def _make_gather(n_rows, seq, vocab, width):
    c8n = seq // 8
    tn = n_rows // 128
    assert tn == _NW
    n_streams = seq
    mesh = plsc.VectorSubcoreMesh(
        core_axis_name="c", subcore_axis_name="s",
        num_cores=_NC, num_subcores=_NS,
    )

    @functools.partial(
        pl.kernel,
        mesh=mesh,
        out_type=jax.ShapeDtypeStruct((seq, 8, tn, 8, 128), jnp.float32),
        scratch_types=[
            pltpu.VMEM((c8n, 8, 128), jnp.int32),
            pltpu.VMEM((2, 128, width), jnp.float32),
            pltpu.VMEM((2, 8, 8, 128), jnp.float32),
            pltpu.SemaphoreType.DMA,
            pltpu.SemaphoreType.DMA,
            pltpu.SemaphoreType.DMA,
            pltpu.SemaphoreType.DMA,
        ],
        compiler_params=pltpu.CompilerParams(
            use_tc_tiling_on_sc=False, needs_layout_passes=False),
    )
    def gather_kernel(table_hbm, tok_hbm, q_hbm, tok_v, rows_v, tr_v,
                      gsem0, gsem1, wsem0, wsem1):
        t = lax.axis_index("s") * _NC + lax.axis_index("c")
        gsems = (gsem0, gsem1)
        wsems = (wsem0, wsem1)

        pltpu.sync_copy(tok_hbm.at[:, t], tok_v)

        iota16 = lax.iota(jnp.int32, 16)
        z16 = jnp.zeros((16,), jnp.int32)
        row_pre = [iota16 + 16 * gl for gl in range(8)]

        def gather_copy(k, nb):
            c8 = lax.div(k, 8)
            s = lax.rem(k, 8)
            return pltpu.make_async_copy(
                table_hbm.at[tok_v.at[c8, s]], rows_v.at[nb], gsems[nb])

        def write_copy(k, nb):
            return pltpu.make_async_copy(
                q_hbm.at[k, :, t], tr_v.at[nb], wsems[nb])

        def transpose(nb):
            @pl.loop(0, 8)
            def _(d8):
                for s_d in range(8):
                    col = z16 + (d8 * 8 + s_d)
                    for gl in range(8):
                        vals = plsc.load_gather(
                            rows_v.at[nb], [row_pre[gl], col])
                        tr_v[nb, d8, s_d, pl.ds(16 * gl, 16)] = vals

        def start_write(k, nb):
            pltpu.async_copy(tr_v.at[nb], q_hbm.at[k, :, t], wsems[nb])

        for nb in range(2):
            gather_copy(nb, nb).start()

        @pl.loop(0, n_streams - 2, step=2)
        def _(k0):
            for nb in range(2):
                k = k0 + nb
                gather_copy(k, nb).wait()

                @pl.when(k0 > 0)
                def _():
                    write_copy(k, nb).wait()

                transpose(nb)
                gather_copy(k + 2, nb).start()
                start_write(k, nb)

        for nb in range(2):
            k = n_streams - 2 + nb
            gather_copy(k, nb).wait()
            write_copy(k, nb).wait()
            transpose(nb)
            start_write(k, nb)
        for nb in range(2):
            write_copy(n_streams - 2 + nb, nb).wait()

    return gather_kernel


@jax.jit
def kernel(tokens, embedding_weight):
    n_rows, seq = tokens.shape
    vocab, width = embedding_weight.shape
    tok_view = (tokens.T.reshape(seq // 8, 8, n_rows // 128, 128)
                .transpose(0, 2, 1, 3))
    q = _make_gather(n_rows, seq, vocab, width)(embedding_weight, tok_view)
    return q.transpose(2, 4, 0, 1, 3).reshape(n_rows, seq, width)


# speedup vs baseline: 1.4622x; 1.4622x over previous
import functools

import jax
import jax.numpy as jnp
from jax import lax
from jax.experimental import pallas as pl
from jax.experimental.pallas import tpu as pltpu
from jax.experimental.pallas import tpu_sc as plsc

_NC = 2
_NS = 16
_NW = _NC * _NS


def _make_gather(n_rows, seq, vocab, width):
    c8n = seq // 8
    tn = n_rows // 128
    assert tn == _NW
    n_streams = seq
    mesh = plsc.VectorSubcoreMesh(
        core_axis_name="c", subcore_axis_name="s",
        num_cores=_NC, num_subcores=_NS,
    )

    @functools.partial(
        pl.kernel,
        mesh=mesh,
        out_type=jax.ShapeDtypeStruct((seq, 8, tn, 8, 128), jnp.float32),
        scratch_types=[
            pltpu.VMEM((c8n, 8, 128), jnp.int32),
            pltpu.VMEM((2, 128, width), jnp.float32),
            pltpu.VMEM((2, 8, 8, 128), jnp.float32),
            pltpu.SemaphoreType.DMA,
            pltpu.SemaphoreType.DMA,
            pltpu.SemaphoreType.DMA,
            pltpu.SemaphoreType.DMA,
        ],
        compiler_params=pltpu.CompilerParams(
            use_tc_tiling_on_sc=False, needs_layout_passes=False),
    )
    def gather_kernel(table_hbm, tok_hbm, q_hbm, tok_v, rows_v, tr_v,
                      gsem0, gsem1, wsem0, wsem1):
        t = lax.axis_index("s") * _NC + lax.axis_index("c")
        gsems = (gsem0, gsem1)
        wsems = (wsem0, wsem1)

        pltpu.sync_copy(tok_hbm.at[:, t], tok_v)

        iota16 = lax.iota(jnp.int32, 16)
        z16 = jnp.zeros((16,), jnp.int32)
        row_pre = [iota16 + 16 * gl for gl in range(8)]

        def gather_copy(k, nb):
            c8 = lax.div(k, 8)
            s = lax.rem(k, 8)
            return pltpu.make_async_copy(
                table_hbm.at[tok_v.at[c8, s]], rows_v.at[nb], gsems[nb])

        def write_copy(k, nb):
            return pltpu.make_async_copy(
                q_hbm.at[k, :, t], tr_v.at[nb], wsems[nb])

        def transpose(nb):
            @plsc.parallel_loop(0, width, unroll=8)
            def _(d):
                col = z16 + d
                d8 = lax.shift_right_logical(d, 3)
                s_d = lax.bitwise_and(d, 7)
                for gl in range(8):
                    vals = plsc.load_gather(
                        rows_v.at[nb], [row_pre[gl], col])
                    tr_v[nb, d8, s_d, pl.ds(16 * gl, 16)] = vals

        def start_write(k, nb):
            pltpu.async_copy(tr_v.at[nb], q_hbm.at[k, :, t], wsems[nb])

        for nb in range(2):
            gather_copy(nb, nb).start()

        @pl.loop(0, n_streams - 2, step=2)
        def _(k0):
            for nb in range(2):
                k = k0 + nb
                gather_copy(k, nb).wait()

                @pl.when(k0 > 0)
                def _():
                    write_copy(k, nb).wait()

                transpose(nb)
                gather_copy(k + 2, nb).start()
                start_write(k, nb)

        for nb in range(2):
            k = n_streams - 2 + nb
            gather_copy(k, nb).wait()
            write_copy(k, nb).wait()
            transpose(nb)
            start_write(k, nb)
        for nb in range(2):
            write_copy(n_streams - 2 + nb, nb).wait()

    return gather_kernel


@jax.jit
def kernel(tokens, embedding_weight):
    n_rows, seq = tokens.shape
    vocab, width = embedding_weight.shape
    tok_view = (tokens.T.reshape(seq // 8, 8, n_rows // 128, 128)
                .transpose(0, 2, 1, 3))
    q = _make_gather(n_rows, seq, vocab, width)(embedding_weight, tok_view)
    return q.transpose(2, 4, 0, 1, 3).reshape(n_rows, seq, width)


# speedup vs baseline: 2.3740x; 1.6236x over previous
import functools

import jax
import jax.numpy as jnp
from jax import lax
from jax.experimental import pallas as pl
from jax.experimental.pallas import tpu as pltpu
from jax.experimental.pallas import tpu_sc as plsc

_NC = 2
_NS = 16
_NW = _NC * _NS


def _make_gather(n_rows, seq, vocab, width):
    c8n = seq // 8
    tn = n_rows // 128
    assert tn == _NW
    n_streams = seq
    mesh = plsc.VectorSubcoreMesh(
        core_axis_name="c", subcore_axis_name="s",
        num_cores=_NC, num_subcores=_NS,
    )

    @functools.partial(
        pl.kernel,
        mesh=mesh,
        out_type=jax.ShapeDtypeStruct((seq, 8, tn, 8, 128), jnp.float32),
        scratch_types=[
            pltpu.VMEM((c8n, 8, 128), jnp.int32),
            pltpu.VMEM((2, 128, width), jnp.float32),
            pltpu.VMEM((2, 64, 128), jnp.float32),
            pltpu.SemaphoreType.DMA,
            pltpu.SemaphoreType.DMA,
            pltpu.SemaphoreType.DMA,
            pltpu.SemaphoreType.DMA,
        ],
        compiler_params=pltpu.CompilerParams(
            use_tc_tiling_on_sc=False, needs_layout_passes=False),
    )
    def gather_kernel(table_hbm, tok_hbm, q_hbm, tok_v, rows_v, tr_v,
                      gsem0, gsem1, wsem0, wsem1):
        t = lax.axis_index("s") * _NC + lax.axis_index("c")
        gsems = (gsem0, gsem1)
        wsems = (wsem0, wsem1)

        pltpu.sync_copy(tok_hbm.at[:, t], tok_v)

        iota16 = lax.iota(jnp.int32, 16)
        diag = [lax.bitwise_and(iota16 + k, 15) for k in range(16)]

        def gather_copy(k, nb):
            c8 = lax.div(k, 8)
            s = lax.rem(k, 8)
            return pltpu.make_async_copy(
                table_hbm.at[tok_v.at[c8, s]], rows_v.at[nb], gsems[nb])

        def write_copies(k, nb):
            return [pltpu.make_async_copy(
                        tr_v.at[nb, pl.ds(a * 8, 8)],
                        q_hbm.at[k, a, t], wsems[nb])
                    for a in range(8)]

        def transpose(nb):
            rows_2d = rows_v.at[nb]
            tr_2d = tr_v.at[nb]

            @plsc.parallel_loop(0, 8 * (width // 16), unroll=2)
            def _(blk):
                l0 = lax.bitwise_and(blk, 7) * 16
                d0 = lax.shift_right_logical(blk, 3) * 16
                lv = iota16 + l0
                for k in range(16):
                    dv = diag[k] + d0
                    vals = plsc.load_gather(rows_2d, [lv, dv])
                    plsc.store_scatter(tr_2d, [dv, lv], vals)

        def start_write(k, nb):
            for d in write_copies(k, nb):
                d.start()

        for nb in range(2):
            gather_copy(nb, nb).start()

        @pl.loop(0, n_streams - 2, step=2)
        def _(k0):
            for nb in range(2):
                k = k0 + nb
                gather_copy(k, nb).wait()

                @pl.when(k0 > 0)
                def _():
                    for d in write_copies(k, nb):
                        d.wait()

                transpose(nb)
                gather_copy(k + 2, nb).start()
                start_write(k, nb)

        for nb in range(2):
            k = n_streams - 2 + nb
            gather_copy(k, nb).wait()
            for d in write_copies(k, nb):
                d.wait()
            transpose(nb)
            start_write(k, nb)
        for nb in range(2):
            for d in write_copies(n_streams - 2 + nb, nb):
                d.wait()

    return gather_kernel


@jax.jit
def kernel(tokens, embedding_weight):
    n_rows, seq = tokens.shape
    vocab, width = embedding_weight.shape
    tok_view = (tokens.T.reshape(seq // 8, 8, n_rows // 128, 128)
                .transpose(0, 2, 1, 3))
    q = _make_gather(n_rows, seq, vocab, width)(embedding_weight, tok_view)
    return q.transpose(2, 4, 0, 1, 3).reshape(n_rows, seq, width)


# speedup vs baseline: 3.5211x; 1.4832x over previous
import functools

import jax
import jax.numpy as jnp
from jax import lax
from jax.experimental import pallas as pl
from jax.experimental.pallas import tpu as pltpu
from jax.experimental.pallas import tpu_sc as plsc

_NC = 2
_NS = 16
_NW = _NC * _NS


def _make_table_detile(vocab, width):
    bv = 4096
    grid = (vocab + bv - 1) // bv

    def body(tt_ref, out_ref):
        out_ref[:, 0:width] = tt_ref[...].T

    return pl.pallas_call(
        body,
        grid=(grid,),
        in_specs=[pl.BlockSpec((width, bv), lambda i: (0, i))],
        out_specs=pl.BlockSpec((bv, 128), lambda i: (i, 0)),
        out_shape=jax.ShapeDtypeStruct((vocab, 128), jnp.float32),
    )


def _make_gather(n_rows, seq, vocab, width):
    c8n = seq // 8
    tn = n_rows // 128
    assert tn == _NW
    n_streams = seq
    mesh = plsc.VectorSubcoreMesh(
        core_axis_name="c", subcore_axis_name="s",
        num_cores=_NC, num_subcores=_NS,
    )

    @functools.partial(
        pl.kernel,
        mesh=mesh,
        out_type=jax.ShapeDtypeStruct((seq, 8, tn, 8, 128), jnp.float32),
        scratch_types=[
            pltpu.VMEM((c8n, 8, 128), jnp.int32),
            pltpu.VMEM((2, 128, width), jnp.float32),
            pltpu.VMEM((2, 64, 128), jnp.float32),
            pltpu.SemaphoreType.DMA,
            pltpu.SemaphoreType.DMA,
            pltpu.SemaphoreType.DMA,
            pltpu.SemaphoreType.DMA,
        ],
        compiler_params=pltpu.CompilerParams(
            use_tc_tiling_on_sc=False, needs_layout_passes=False),
    )
    def gather_kernel(table_hbm, tok_hbm, q_hbm, tok_v, rows_v, tr_v,
                      gsem0, gsem1, wsem0, wsem1):
        t = lax.axis_index("s") * _NC + lax.axis_index("c")
        gsems = (gsem0, gsem1)
        wsems = (wsem0, wsem1)

        pltpu.sync_copy(tok_hbm.at[:, t], tok_v)

        @plsc.parallel_loop(0, c8n * 8 * 8, unroll=8)
        def _(i):
            c8 = lax.shift_right_logical(i, 6)
            s = lax.bitwise_and(lax.shift_right_logical(i, 3), 7)
            g = lax.bitwise_and(i, 7) * 16
            tok_v[c8, s, pl.ds(g, 16)] = tok_v[c8, s, pl.ds(g, 16)] * 2

        iota16 = lax.iota(jnp.int32, 16)
        diag = [lax.bitwise_and(iota16 + k, 15) for k in range(16)]

        def gather_copy(k, nb):
            c8 = lax.div(k, 8)
            s = lax.rem(k, 8)
            return pltpu.make_async_copy(
                table_hbm.at[tok_v.at[c8, s]], rows_v.at[nb], gsems[nb])

        def write_copies(k, nb):
            return [pltpu.make_async_copy(
                        tr_v.at[nb, pl.ds(a * 8, 8)],
                        q_hbm.at[k, a, t], wsems[nb])
                    for a in range(8)]

        def transpose(nb):
            rows_2d = rows_v.at[nb]
            tr_2d = tr_v.at[nb]

            @plsc.parallel_loop(0, 8 * (width // 16), unroll=2)
            def _(blk):
                l0 = lax.bitwise_and(blk, 7) * 16
                d0 = lax.shift_right_logical(blk, 3) * 16
                lv = iota16 + l0
                for k in range(16):
                    dv = diag[k] + d0
                    vals = plsc.load_gather(rows_2d, [lv, dv])
                    plsc.store_scatter(tr_2d, [dv, lv], vals)

        def start_write(k, nb):
            for d in write_copies(k, nb):
                d.start()

        for nb in range(2):
            gather_copy(nb, nb).start()

        @pl.loop(0, n_streams - 2, step=2)
        def _(k0):
            for nb in range(2):
                k = k0 + nb
                gather_copy(k, nb).wait()

                @pl.when(k0 > 0)
                def _():
                    for d in write_copies(k, nb):
                        d.wait()

                transpose(nb)
                gather_copy(k + 2, nb).start()
                start_write(k, nb)

        for nb in range(2):
            k = n_streams - 2 + nb
            gather_copy(k, nb).wait()
            for d in write_copies(k, nb):
                d.wait()
            transpose(nb)
            start_write(k, nb)
        for nb in range(2):
            for d in write_copies(n_streams - 2 + nb, nb):
                d.wait()

    return gather_kernel


@jax.jit
def kernel(tokens, embedding_weight):
    n_rows, seq = tokens.shape
    vocab, width = embedding_weight.shape
    tok_view = (tokens.T.reshape(seq // 8, 8, n_rows // 128, 128)
                .transpose(0, 2, 1, 3))
    w128 = _make_table_detile(vocab, width)(embedding_weight.T)
    q = _make_gather(n_rows, seq, vocab, width)(
        w128.reshape(2 * vocab, width), tok_view)
    return q.transpose(2, 4, 0, 1, 3).reshape(n_rows, seq, width)


# speedup vs baseline: 4.1464x; 1.1776x over previous
import functools

import jax
import jax.numpy as jnp
from jax import lax
from jax.experimental import pallas as pl
from jax.experimental.pallas import tpu as pltpu
from jax.experimental.pallas import tpu_sc as plsc

_NC = 2
_NS = 16
_NW = _NC * _NS


_BV = 4096


def _make_table_detile(vocab, width):
    pairs = (vocab + 2 * _BV - 1) // (2 * _BV)

    def body(a_ref, b_ref, out_ref):
        out_ref[:, 0:width] = a_ref[...].T
        out_ref[:, width:2 * width] = b_ref[...].T

    return pl.pallas_call(
        body,
        grid=(pairs,),
        in_specs=[pl.BlockSpec((width, _BV), lambda i: (0, 2 * i)),
                  pl.BlockSpec(
                      (width, _BV),
                      lambda i: (0, jnp.minimum(2 * i + 1,
                                                (vocab - 1) // _BV)))],
        out_specs=pl.BlockSpec((_BV, 2 * width), lambda i: (i, 0)),
        out_shape=jax.ShapeDtypeStruct((pairs * _BV, 2 * width),
                                       jnp.float32),
    )


def _make_gather(n_rows, seq, vocab, width):
    c8n = seq // 8
    tn = n_rows // 128
    assert tn == _NW
    n_streams = seq
    mesh = plsc.VectorSubcoreMesh(
        core_axis_name="c", subcore_axis_name="s",
        num_cores=_NC, num_subcores=_NS,
    )

    @functools.partial(
        pl.kernel,
        mesh=mesh,
        out_type=jax.ShapeDtypeStruct((seq, 8, tn, 8, 128), jnp.float32),
        scratch_types=[
            pltpu.VMEM((c8n, 8, 128), jnp.int32),
            pltpu.VMEM((2, 128, width), jnp.float32),
            pltpu.VMEM((2, 64, 128), jnp.float32),
            pltpu.SemaphoreType.DMA,
            pltpu.SemaphoreType.DMA,
            pltpu.SemaphoreType.DMA,
            pltpu.SemaphoreType.DMA,
        ],
        compiler_params=pltpu.CompilerParams(
            use_tc_tiling_on_sc=False, needs_layout_passes=False),
    )
    def gather_kernel(table_hbm, tok_hbm, q_hbm, tok_v, rows_v, tr_v,
                      gsem0, gsem1, wsem0, wsem1):
        t = lax.axis_index("s") * _NC + lax.axis_index("c")
        gsems = (gsem0, gsem1)
        wsems = (wsem0, wsem1)

        pltpu.sync_copy(tok_hbm.at[:, t], tok_v)

        @plsc.parallel_loop(0, c8n * 8 * 8, unroll=8)
        def _(i):
            c8 = lax.shift_right_logical(i, 6)
            s = lax.bitwise_and(lax.shift_right_logical(i, 3), 7)
            g = lax.bitwise_and(i, 7) * 16
            v = tok_v[c8, s, pl.ds(g, 16)]
            idx = (lax.shift_left(lax.shift_right_logical(v, 13), 13)
                   + lax.shift_left(lax.bitwise_and(v, _BV - 1), 1)
                   + lax.bitwise_and(lax.shift_right_logical(v, 12), 1))
            tok_v[c8, s, pl.ds(g, 16)] = idx

        iota16 = lax.iota(jnp.int32, 16)
        diag = [lax.bitwise_and(iota16 + k, 15) for k in range(16)]

        def gather_copy(k, nb):
            c8 = lax.div(k, 8)
            s = lax.rem(k, 8)
            return pltpu.make_async_copy(
                table_hbm.at[tok_v.at[c8, s]], rows_v.at[nb], gsems[nb])

        def write_copies(k, nb):
            return [pltpu.make_async_copy(
                        tr_v.at[nb, pl.ds(a * 8, 8)],
                        q_hbm.at[k, a, t], wsems[nb])
                    for a in range(8)]

        def transpose(nb):
            rows_2d = rows_v.at[nb]
            tr_2d = tr_v.at[nb]

            @plsc.parallel_loop(0, 8 * (width // 16), unroll=2)
            def _(blk):
                l0 = lax.bitwise_and(blk, 7) * 16
                d0 = lax.shift_right_logical(blk, 3) * 16
                lv = iota16 + l0
                for k in range(16):
                    dv = diag[k] + d0
                    vals = plsc.load_gather(rows_2d, [lv, dv])
                    plsc.store_scatter(tr_2d, [dv, lv], vals)

        def start_write(k, nb):
            for d in write_copies(k, nb):
                d.start()

        for nb in range(2):
            gather_copy(nb, nb).start()

        @pl.loop(0, n_streams - 2, step=2)
        def _(k0):
            for nb in range(2):
                k = k0 + nb
                gather_copy(k, nb).wait()

                @pl.when(k0 > 0)
                def _():
                    for d in write_copies(k, nb):
                        d.wait()

                transpose(nb)
                gather_copy(k + 2, nb).start()
                start_write(k, nb)

        for nb in range(2):
            k = n_streams - 2 + nb
            gather_copy(k, nb).wait()
            for d in write_copies(k, nb):
                d.wait()
            transpose(nb)
            start_write(k, nb)
        for nb in range(2):
            for d in write_copies(n_streams - 2 + nb, nb):
                d.wait()

    return gather_kernel


@jax.jit
def kernel(tokens, embedding_weight):
    n_rows, seq = tokens.shape
    vocab, width = embedding_weight.shape
    tok_view = (tokens.T.reshape(seq // 8, 8, n_rows // 128, 128)
                .transpose(0, 2, 1, 3))
    tt = embedding_weight.T
    w2 = _make_table_detile(vocab, width)(tt, tt)
    q = _make_gather(n_rows, seq, vocab, width)(
        w2.reshape(w2.shape[0] * 2, width), tok_view)
    return q.transpose(2, 4, 0, 1, 3).reshape(n_rows, seq, width)
